# baseline (speedup 1.0000x reference)
import functools

import jax
import jax.numpy as jnp
from jax import lax
from jax.experimental import pallas as pl
from jax.experimental.pallas import tpu as pltpu
from jax.experimental.pallas import tpu_sc as plsc

BATCH = 64
BEAM = 8
VOCAB = 100000
ROW = BEAM * VOCAB
K = 8
L = 16
NC, NS = 2, 16
NW = NC * NS
ROWS_PER_W = BATCH // NW
CHUNK = 32000
NCHUNK = ROW // CHUNK
BVREGS = 16
BLOCK = BVREGS * L
NBLOCK = CHUNK // BLOCK
GB = 1
NGROUP = NBLOCK // GB
WARM_BLOCKS = GB
CAP = 4096
CAP_SAFE = CAP - 64

assert CHUNK % BLOCK == 0 and NBLOCK % GB == 0

NEG_INF = float("-inf")
I32_MAX = 2**31 - 1


def _bubble_insert(v, iv, vals, idxs):
    vals = list(vals)
    idxs = list(idxs)
    nv, ni = v, iv
    for lvl in range(K):
        tv, ti = vals[lvl], idxs[lvl]
        take = (nv > tv) | ((nv == tv) & (ni < ti))
        vals[lvl] = jnp.where(take, nv, tv)
        idxs[lvl] = jnp.where(take, ni, ti)
        nv = jnp.where(take, tv, nv)
        ni = jnp.where(take, ti, ni)
    return tuple(vals), tuple(idxs)


def _tec_body(score_hbm, vals_hbm, rows_hbm, cols_hbm,
              buf_a, buf_b, maxbuf, cv, ci, vbuf, ibuf, wl,
              ov_ref, or_ref, oc_ref, sem_a, sem_b):
    wid = lax.axis_index("s") * NC + lax.axis_index("c")
    lane = lax.iota(jnp.int32, L)

    def raw_append(c, buf, vreg0, nvregs, off0):
        def rbody(i, off):
            v = buf[pl.ds((vreg0 + i) * L, L)]
            iv = lane + (c * CHUNK + (vreg0 + i) * L)
            cv[pl.ds(off, L)] = v
            ci[pl.ds(off, L)] = iv
            return off + L

        return lax.fori_loop(0, nvregs, rbody, off0)

    def drain(off):
        for t in range(4):
            cv[pl.ds(off + t * L, L)] = jnp.full((L,), NEG_INF, jnp.float32)
            ci[pl.ds(off + t * L, L)] = jnp.full((L,), 0, jnp.int32)
        vals = tuple(vbuf[pl.ds(l * L, L)] for l in range(K))
        idxs = tuple(ibuf[pl.ds(l * L, L)] for l in range(K))

        def dbody(j, vi):
            vals, idxs = vi
            for t in range(4):
                vals, idxs = _bubble_insert(cv[pl.ds(j * 4 * L + t * L, L)],
                                            ci[pl.ds(j * 4 * L + t * L, L)],
                                            vals, idxs)
            return vals, idxs

        vals, idxs = lax.fori_loop(0, (off + 4 * L - 1) // (4 * L), dbody,
                                   (vals, idxs))
        for l in range(K):
            vbuf[pl.ds(l * L, L)] = vals[l]
            ibuf[pl.ds(l * L, L)] = idxs[l]
        return jnp.maximum(vals[K - 1],
                           jnp.full((L,), jnp.min(vals[0]), jnp.float32))

    def append_block(c, b, buf, t8, offv):
        for i in range(BVREGS):
            v = buf[pl.ds(b * BLOCK + i * L, L)]
            m = v >= t8
            mi = jnp.where(m, jnp.int32(1), jnp.int32(0))
            cs = plsc.cumsum(mi)
            pos = jnp.minimum(offv + cs - mi, CAP - 1)
            iv = lane + (c * CHUNK + b * BLOCK + i * L)
            plsc.store_scatter(cv, [pos], v, mask=m)
            plsc.store_scatter(ci, [pos], iv, mask=m)
            offv = offv + plsc.all_reduce_population_count(m)
        return offv

    def process_chunk(c, buf, t8, first):
        def pass1(b):
            base = b * BLOCK
            acc = buf[pl.ds(base, L)]
            for i in range(1, BVREGS):
                acc = jnp.maximum(acc, buf[pl.ds(base + i * L, L)])
            maxbuf[pl.ds(b * L, L)] = acc

        plsc.parallel_loop(0, NBLOCK, unroll=4)(pass1)

        def scan_group(j, wo, nb, skip):
            hitc = jnp.full((L,), 0, jnp.int32)
            for g in range(nb):
                mg = maxbuf[pl.ds((j * L + g) * L, L)] >= t8
                cg = plsc.all_reduce_population_count(mg)
                hitc = jnp.where(lane == g, cg, hitc)
            m2 = hitc > 0
            if nb < L:
                m2 = m2 & (lane < nb)
            if skip > 0:
                m2 = m2 & (lane >= skip)
            bids = lane + j * L
            plsc.store_compressed(wl.at[pl.ds(wo, L)], bids, mask=m2)
            return wo + plsc.all_reduce_population_count(m2)[0]

        wo = jnp.int32(0)
        for j in range(NBLOCK // L):
            wo = scan_group(j, wo, L, WARM_BLOCKS if (first and j == 0) else 0)
        if NBLOCK % L:
            wo = scan_group(NBLOCK // L, wo, NBLOCK % L, 0)

        def have_work(t8):
            def wbody(e, offv):
                b = wl[pl.ds(e, L)][0]
                return append_block(c, b, buf, t8, offv)

            offv = lax.fori_loop(0, wo, wbody, jnp.zeros((L,), jnp.int32))
            off = offv[0]

            def recover():
                def rbody(g, t8):
                    o = raw_append(c, buf, g * GB * BVREGS, GB * BVREGS,
                                   jnp.int32(0))
                    return drain(o)

                return lax.fori_loop(1 if first else 0, NGROUP, rbody, t8)

            return lax.cond(off > CAP_SAFE, recover, lambda: drain(off))

        return lax.cond(wo > 0, have_work, lambda t: t, t8)

    def row_body(rr, out):
        out_v, out_i = out
        row_base = (wid * ROWS_PER_W + rr) * ROW

        def copy_into(ch, buf, sem):
            return pltpu.make_async_copy(
                score_hbm.at[pl.ds(row_base + ch * CHUNK, CHUNK)], buf, sem)

        for l in range(K):
            vbuf[pl.ds(l * L, L)] = jnp.full((L,), NEG_INF, jnp.float32)
            ibuf[pl.ds(l * L, L)] = jnp.full((L,), 0, jnp.int32)

        copy_into(0, buf_a, sem_a).start()
        copy_into(1, buf_b, sem_b).start()
        copy_into(0, buf_a, sem_a).wait()

        t8 = drain(raw_append(0, buf_a, 0, WARM_BLOCKS * BVREGS,
                              jnp.int32(0)))
        t8 = process_chunk(0, buf_a, t8, first=True)

        def pair_body(p, t8):
            ca = 2 * p + 1
            copy_into(ca + 1, buf_a, sem_a).start()
            copy_into(ca, buf_b, sem_b).wait()
            t8 = process_chunk(ca, buf_b, t8, first=False)
            copy_into(jnp.minimum(ca + 2, NCHUNK - 1), buf_b, sem_b).start()
            copy_into(ca + 1, buf_a, sem_a).wait()
            return process_chunk(ca + 1, buf_a, t8, first=False)

        t8 = lax.fori_loop(0, (NCHUNK - 1) // 2, pair_body, t8)
        copy_into(NCHUNK - 1, buf_b, sem_b).wait()

        vals = [vbuf[pl.ds(l * L, L)] for l in range(K)]
        idxs = [ibuf[pl.ds(l * L, L)] for l in range(K)]

        for p in range(K):
            mv = vals[0]
            for j in range(1, K):
                mv = jnp.maximum(mv, vals[j])
            m = jnp.max(mv)
            iw = [jnp.where(vals[j] == m, idxs[j], I32_MAX) for j in range(K)]
            mi = iw[0]
            for j in range(1, K):
                mi = jnp.minimum(mi, iw[j])
            mi = jnp.min(mi)
            for j in range(K):
                vals[j] = jnp.where((vals[j] == m) & (idxs[j] == mi),
                                    NEG_INF, vals[j])
            sel = lane == (rr * K + p)
            out_v = jnp.where(sel, m, out_v)
            out_i = jnp.where(sel, mi, out_i)
        return out_v, out_i

    out_v = jnp.full((L,), 0.0, jnp.float32)
    out_i = jnp.full((L,), 0, jnp.int32)
    out_v, out_i = lax.fori_loop(0, ROWS_PER_W, row_body, (out_v, out_i))

    out_r = out_i // VOCAB
    out_c = out_i - out_r * VOCAB
    ov_ref[...] = out_v
    or_ref[...] = out_r
    oc_ref[...] = out_c
    pltpu.sync_copy(ov_ref, vals_hbm.at[pl.ds(wid * L, L)])
    pltpu.sync_copy(or_ref, rows_hbm.at[pl.ds(wid * L, L)])
    pltpu.sync_copy(oc_ref, cols_hbm.at[pl.ds(wid * L, L)])


@jax.jit
def kernel(score):
    flat = score.reshape(BATCH * ROW)
    mesh = plsc.VectorSubcoreMesh(core_axis_name="c", subcore_axis_name="s",
                                  num_cores=NC, num_subcores=NS)
    vals, rows, cols = pl.kernel(
        _tec_body,
        out_type=(
            jax.ShapeDtypeStruct((BATCH * K,), jnp.float32),
            jax.ShapeDtypeStruct((BATCH * K,), jnp.int32),
            jax.ShapeDtypeStruct((BATCH * K,), jnp.int32),
        ),
        mesh=mesh,
        compiler_params=pltpu.CompilerParams(needs_layout_passes=False),
        scratch_types=[
            pltpu.VMEM((CHUNK,), jnp.float32),
            pltpu.VMEM((CHUNK,), jnp.float32),
            pltpu.VMEM((NBLOCK * L,), jnp.float32),
            pltpu.VMEM((CAP,), jnp.float32),
            pltpu.VMEM((CAP,), jnp.int32),
            pltpu.VMEM((K * L,), jnp.float32),
            pltpu.VMEM((K * L,), jnp.int32),
            pltpu.VMEM((NBLOCK + L,), jnp.int32),
            pltpu.VMEM((L,), jnp.float32),
            pltpu.VMEM((L,), jnp.int32),
            pltpu.VMEM((L,), jnp.int32),
            pltpu.SemaphoreType.DMA,
            pltpu.SemaphoreType.DMA,
        ],
    )(flat)
    return (vals.reshape(BATCH, K), rows.reshape(BATCH, K),
            cols.reshape(BATCH, K))


# speedup vs baseline: 1.3666x; 1.3666x over previous
import functools

import jax
import jax.numpy as jnp
from jax import lax
from jax.experimental import pallas as pl
from jax.experimental.pallas import tpu as pltpu
from jax.experimental.pallas import tpu_sc as plsc

BATCH = 64
BEAM = 8
VOCAB = 100000
ROW = BEAM * VOCAB
K = 8
L = 16
NC, NS = 2, 16
NW = NC * NS
ROWS_PER_W = BATCH // NW
CHUNK = 32000
NCHUNK = ROW // CHUNK
BVREGS = 16
BLOCK = BVREGS * L
NBLOCK = CHUNK // BLOCK
GB = 5
NGROUP = NBLOCK // GB
WARM_BLOCKS = GB
CAP = 4096
CAP_SAFE = CAP - 16

assert CHUNK % BLOCK == 0 and NBLOCK % GB == 0

NEG_INF = float("-inf")
I32_MAX = 2**31 - 1


def _bubble_insert(v, iv, vals, idxs):
    vals = list(vals)
    idxs = list(idxs)
    nv, ni = v, iv
    for lvl in range(K):
        tv, ti = vals[lvl], idxs[lvl]
        take = (nv > tv) | ((nv == tv) & (ni < ti))
        vals[lvl] = jnp.where(take, nv, tv)
        idxs[lvl] = jnp.where(take, ni, ti)
        nv = jnp.where(take, tv, nv)
        ni = jnp.where(take, ti, ni)
    return tuple(vals), tuple(idxs)


def _tec_body(score_hbm, vals_hbm, rows_hbm, cols_hbm,
              buf_a, buf_b, maxbuf, cv, ci, vbuf, ibuf, wl,
              ov_ref, or_ref, oc_ref, sem_a, sem_b):
    wid = lax.axis_index("s") * NC + lax.axis_index("c")
    lane = lax.iota(jnp.int32, L)

    def raw_append(c, buf, vreg0, nvregs, off0):
        def rbody(i, off):
            v = buf[pl.ds((vreg0 + i) * L, L)]
            iv = lane + (c * CHUNK + (vreg0 + i) * L)
            cv[pl.ds(off, L)] = v
            ci[pl.ds(off, L)] = iv
            return off + L

        return lax.fori_loop(0, nvregs, rbody, off0)

    def drain(off):
        cv[pl.ds(off, L)] = jnp.full((L,), NEG_INF, jnp.float32)
        ci[pl.ds(off, L)] = jnp.full((L,), 0, jnp.int32)
        vals = tuple(vbuf[pl.ds(l * L, L)] for l in range(K))
        idxs = tuple(ibuf[pl.ds(l * L, L)] for l in range(K))

        def dbody(j, vi):
            return _bubble_insert(cv[pl.ds(j * L, L)], ci[pl.ds(j * L, L)],
                                  *vi)

        vals, idxs = lax.fori_loop(0, (off + L - 1) // L, dbody, (vals, idxs))
        for l in range(K):
            vbuf[pl.ds(l * L, L)] = vals[l]
            ibuf[pl.ds(l * L, L)] = idxs[l]
        cvals = list(vals)
        for p in range(K):
            mv = cvals[0]
            for j in range(1, K):
                mv = jnp.maximum(mv, cvals[j])
            g8 = jnp.max(mv)
            if p < K - 1:
                for j in range(K):
                    cvals[j] = jnp.where(cvals[j] == g8, NEG_INF, cvals[j])
        return jnp.maximum(vals[K - 1], jnp.full((L,), g8, jnp.float32))

    def append_block(c, b, buf, t8, offv):
        for i in range(BVREGS):
            v = buf[pl.ds(b * BLOCK + i * L, L)]
            m = v >= t8
            mi = jnp.where(m, jnp.int32(1), jnp.int32(0))
            cs = plsc.cumsum(mi)
            pos = jnp.minimum(offv + cs - mi, CAP - 1)
            iv = lane + (c * CHUNK + b * BLOCK + i * L)
            plsc.store_scatter(cv, [pos], v, mask=m)
            plsc.store_scatter(ci, [pos], iv, mask=m)
            offv = offv + plsc.all_reduce_population_count(m)
        return offv

    def process_chunk(c, buf, t8, first):
        def pass1(b):
            base = b * BLOCK
            acc = buf[pl.ds(base, L)]
            for i in range(1, BVREGS):
                acc = jnp.maximum(acc, buf[pl.ds(base + i * L, L)])
            maxbuf[pl.ds(b * L, L)] = acc

        plsc.parallel_loop(0, NBLOCK, unroll=4)(pass1)

        def scan_group(j, wo, nb, skip):
            hitc = jnp.full((L,), 0, jnp.int32)
            for g in range(nb):
                mg = maxbuf[pl.ds((j * L + g) * L, L)] >= t8
                cg = plsc.all_reduce_population_count(mg)
                hitc = jnp.where(lane == g, cg, hitc)
            m2 = hitc > 0
            if nb < L:
                m2 = m2 & (lane < nb)
            if skip > 0:
                m2 = m2 & (lane >= skip)
            bids = lane + j * L
            plsc.store_compressed(wl.at[pl.ds(wo, L)], bids, mask=m2)
            return wo + plsc.all_reduce_population_count(m2)[0]

        wo = jnp.int32(0)
        for j in range(NBLOCK // L):
            wo = scan_group(j, wo, L, WARM_BLOCKS if (first and j == 0) else 0)
        if NBLOCK % L:
            wo = scan_group(NBLOCK // L, wo, NBLOCK % L, 0)

        def have_work(t8):
            def wbody(e, offv):
                b = wl[pl.ds(e, L)][0]
                return append_block(c, b, buf, t8, offv)

            offv = lax.fori_loop(0, wo, wbody, jnp.zeros((L,), jnp.int32))
            off = offv[0]

            def recover():
                def rbody(g, t8):
                    o = raw_append(c, buf, g * GB * BVREGS, GB * BVREGS,
                                   jnp.int32(0))
                    return drain(o)

                return lax.fori_loop(1 if first else 0, NGROUP, rbody, t8)

            return lax.cond(off > CAP_SAFE, recover, lambda: drain(off))

        return lax.cond(wo > 0, have_work, lambda t: t, t8)

    def row_body(rr, out):
        out_v, out_i = out
        row_base = (wid * ROWS_PER_W + rr) * ROW

        def copy_into(ch, buf, sem):
            return pltpu.make_async_copy(
                score_hbm.at[pl.ds(row_base + ch * CHUNK, CHUNK)], buf, sem)

        for l in range(K):
            vbuf[pl.ds(l * L, L)] = jnp.full((L,), NEG_INF, jnp.float32)
            ibuf[pl.ds(l * L, L)] = jnp.full((L,), 0, jnp.int32)

        copy_into(0, buf_a, sem_a).start()
        copy_into(1, buf_b, sem_b).start()
        copy_into(0, buf_a, sem_a).wait()

        t8 = drain(raw_append(0, buf_a, 0, WARM_BLOCKS * BVREGS,
                              jnp.int32(0)))
        t8 = process_chunk(0, buf_a, t8, first=True)

        def pair_body(p, t8):
            ca = 2 * p + 1
            copy_into(ca + 1, buf_a, sem_a).start()
            copy_into(ca, buf_b, sem_b).wait()
            t8 = process_chunk(ca, buf_b, t8, first=False)
            copy_into(jnp.minimum(ca + 2, NCHUNK - 1), buf_b, sem_b).start()
            copy_into(ca + 1, buf_a, sem_a).wait()
            return process_chunk(ca + 1, buf_a, t8, first=False)

        t8 = lax.fori_loop(0, (NCHUNK - 1) // 2, pair_body, t8)
        copy_into(NCHUNK - 1, buf_b, sem_b).wait()

        vals = [vbuf[pl.ds(l * L, L)] for l in range(K)]
        idxs = [ibuf[pl.ds(l * L, L)] for l in range(K)]

        for p in range(K):
            mv = vals[0]
            for j in range(1, K):
                mv = jnp.maximum(mv, vals[j])
            m = jnp.max(mv)
            iw = [jnp.where(vals[j] == m, idxs[j], I32_MAX) for j in range(K)]
            mi = iw[0]
            for j in range(1, K):
                mi = jnp.minimum(mi, iw[j])
            mi = jnp.min(mi)
            for j in range(K):
                vals[j] = jnp.where((vals[j] == m) & (idxs[j] == mi),
                                    NEG_INF, vals[j])
            sel = lane == (rr * K + p)
            out_v = jnp.where(sel, m, out_v)
            out_i = jnp.where(sel, mi, out_i)
        return out_v, out_i

    out_v = jnp.full((L,), 0.0, jnp.float32)
    out_i = jnp.full((L,), 0, jnp.int32)
    out_v, out_i = lax.fori_loop(0, ROWS_PER_W, row_body, (out_v, out_i))

    out_r = out_i // VOCAB
    out_c = out_i - out_r * VOCAB
    ov_ref[...] = out_v
    or_ref[...] = out_r
    oc_ref[...] = out_c
    pltpu.sync_copy(ov_ref, vals_hbm.at[pl.ds(wid * L, L)])
    pltpu.sync_copy(or_ref, rows_hbm.at[pl.ds(wid * L, L)])
    pltpu.sync_copy(oc_ref, cols_hbm.at[pl.ds(wid * L, L)])


@jax.jit
def kernel(score):
    flat = score.reshape(BATCH * ROW)
    mesh = plsc.VectorSubcoreMesh(core_axis_name="c", subcore_axis_name="s",
                                  num_cores=NC, num_subcores=NS)
    vals, rows, cols = pl.kernel(
        _tec_body,
        out_type=(
            jax.ShapeDtypeStruct((BATCH * K,), jnp.float32),
            jax.ShapeDtypeStruct((BATCH * K,), jnp.int32),
            jax.ShapeDtypeStruct((BATCH * K,), jnp.int32),
        ),
        mesh=mesh,
        compiler_params=pltpu.CompilerParams(needs_layout_passes=False),
        scratch_types=[
            pltpu.VMEM((CHUNK,), jnp.float32),
            pltpu.VMEM((CHUNK,), jnp.float32),
            pltpu.VMEM((NBLOCK * L,), jnp.float32),
            pltpu.VMEM((CAP,), jnp.float32),
            pltpu.VMEM((CAP,), jnp.int32),
            pltpu.VMEM((K * L,), jnp.float32),
            pltpu.VMEM((K * L,), jnp.int32),
            pltpu.VMEM((NBLOCK + L,), jnp.int32),
            pltpu.VMEM((L,), jnp.float32),
            pltpu.VMEM((L,), jnp.int32),
            pltpu.VMEM((L,), jnp.int32),
            pltpu.SemaphoreType.DMA,
            pltpu.SemaphoreType.DMA,
        ],
    )(flat)
    return (vals.reshape(BATCH, K), rows.reshape(BATCH, K),
            cols.reshape(BATCH, K))
